# Initial kernel scaffold; baseline (speedup 1.0000x reference)
import jax
import jax.numpy as jnp
from jax.experimental import pallas as pl


def kernel(inputs, targets):
    raise NotImplementedError("write your pallas kernel here")


# speedup vs baseline: 41.8513x; 41.8513x over previous
import functools

import jax
import jax.numpy as jnp
from jax import lax
from jax.experimental import pallas as pl
from jax.experimental.pallas import tpu as pltpu

_W = (0.8373, 0.918, 0.866, 1.0345, 1.0166, 0.9969, 0.9754, 1.0489,
      0.8786, 1.0023, 0.9539, 0.9843, 1.1116, 0.9037, 1.0865, 1.0955,
      1.0865, 1.1529, 1.0507)
_THRESH = 0.7
_MIN_KEPT = 100000
_C = 19
_BH = 64


def _pixel_stats(x, t):
    m = jnp.max(x, axis=0)
    sh = x - m[None]
    s = jnp.sum(jnp.exp(sh), axis=0)
    st = jnp.zeros_like(m)
    wt = jnp.zeros_like(m)
    for c in range(_C):
        sel = t == c
        st = jnp.where(sel, sh[c], st)
        wt = jnp.where(sel, jnp.float32(_W[c]), wt)
    return st, jnp.log(s), wt


def _fused_body(pred_ref, tgt_ref, cnt_ref, a_ref, b_ref):
    x = pred_ref[0]
    t = tgt_ref[0]
    st, logs, wt = _pixel_stats(x, t)
    nll = logs - st
    p = jnp.exp(st) / jnp.exp(logs)
    kept = p <= jnp.float32(_THRESH)
    cnt_p = jnp.sum(kept.astype(jnp.float32))
    a_p = jnp.sum(jnp.where(kept, wt * nll, 0.0))
    b_p = jnp.sum(jnp.where(kept, wt, 0.0))

    first = jnp.logical_and(pl.program_id(0) == 0, pl.program_id(1) == 0)

    @pl.when(first)
    def _init():
        cnt_ref[0, 0] = jnp.float32(0)
        a_ref[0, 0] = jnp.float32(0)
        b_ref[0, 0] = jnp.float32(0)

    cnt_ref[0, 0] += cnt_p
    a_ref[0, 0] += a_p
    b_ref[0, 0] += b_p


def _fallback_arrays_body(pred_ref, tgt_ref, p_ref, nll_ref, wt_ref):
    x = pred_ref[0]
    t = tgt_ref[0]
    st, logs, wt = _pixel_stats(x, t)
    p_ref[0] = jnp.exp(st) / jnp.exp(logs)
    nll_ref[0] = logs - st
    wt_ref[0] = wt


def _fallback_reduce_body(p_ref, nll_ref, wt_ref, thr_ref, num_ref, den_ref):
    p = p_ref[...]
    u = lax.bitcast_convert_type(p, jnp.int32)

    def step(_, carry):
        lo, hi = carry
        mid = (lo + hi) // 2
        c = jnp.sum((u <= mid).astype(jnp.float32))
        take = c >= jnp.float32(_MIN_KEPT)
        return jnp.where(take, lo, mid + 1), jnp.where(take, mid, hi)

    lo0 = jnp.int32(0)
    hi0 = jnp.int32(0x7F800000)
    _, hi = lax.fori_loop(0, 31, step, (lo0, hi0))
    tval = lax.bitcast_convert_type(hi, jnp.float32)
    thr = jnp.maximum(tval, jnp.float32(_THRESH))
    kept = p <= thr
    w = jnp.where(kept, wt_ref[...], 0.0)
    num_ref[0, 0] = jnp.sum(w * nll_ref[...])
    den_ref[0, 0] = jnp.sum(w)
    thr_ref[0, 0] = thr


def _hot_path(pred, tgt):
    n, c, h, w = pred.shape
    grid = (n, h // _BH)
    out = pl.pallas_call(
        _fused_body,
        grid=grid,
        in_specs=[
            pl.BlockSpec((1, c, _BH, w), lambda i, j: (i, 0, j, 0)),
            pl.BlockSpec((1, _BH, w), lambda i, j: (i, j, 0)),
        ],
        out_specs=[
            pl.BlockSpec((1, 1), lambda i, j: (0, 0),
                         memory_space=pltpu.SMEM),
            pl.BlockSpec((1, 1), lambda i, j: (0, 0),
                         memory_space=pltpu.SMEM),
            pl.BlockSpec((1, 1), lambda i, j: (0, 0),
                         memory_space=pltpu.SMEM),
        ],
        out_shape=[jax.ShapeDtypeStruct((1, 1), jnp.float32)] * 3,
    )(pred, tgt)
    return out[0][0, 0], out[1][0, 0], out[2][0, 0]


def _fallback(pred, tgt):
    n, c, h, w = pred.shape
    grid = (n, h // _BH)
    px_shape = jax.ShapeDtypeStruct((n, h, w), jnp.float32)
    p, nll, wt = pl.pallas_call(
        _fallback_arrays_body,
        grid=grid,
        in_specs=[
            pl.BlockSpec((1, c, _BH, w), lambda i, j: (i, 0, j, 0)),
            pl.BlockSpec((1, _BH, w), lambda i, j: (i, j, 0)),
        ],
        out_specs=[pl.BlockSpec((1, _BH, w), lambda i, j: (i, j, 0))] * 3,
        out_shape=[px_shape] * 3,
    )(pred, tgt)
    thr, num, den = pl.pallas_call(
        _fallback_reduce_body,
        out_specs=[
            pl.BlockSpec(memory_space=pltpu.SMEM),
            pl.BlockSpec(memory_space=pltpu.SMEM),
            pl.BlockSpec(memory_space=pltpu.SMEM),
        ],
        out_shape=[jax.ShapeDtypeStruct((1, 1), jnp.float32)] * 3,
    )(p, nll, wt)
    return num[0, 0], den[0, 0]


@jax.jit
def kernel(inputs, targets):
    pred = inputs.astype(jnp.float32)
    tgt = targets.astype(jnp.int32)
    cnt, a, b = _hot_path(pred, tgt)
    num, den = lax.cond(
        cnt >= jnp.float32(_MIN_KEPT),
        lambda: (a, b),
        lambda: _fallback(pred, tgt),
    )
    return num / den


# speedup vs baseline: 42.2786x; 1.0102x over previous
import functools

import jax
import jax.numpy as jnp
from jax import lax
from jax.experimental import pallas as pl
from jax.experimental.pallas import tpu as pltpu

_W = (0.8373, 0.918, 0.866, 1.0345, 1.0166, 0.9969, 0.9754, 1.0489,
      0.8786, 1.0023, 0.9539, 0.9843, 1.1116, 0.9037, 1.0865, 1.0955,
      1.0865, 1.1529, 1.0507)
_THRESH = 0.7
_MIN_KEPT = 100000
_C = 19
_BH = 64


def _pixel_stats(x, t):
    m = jnp.max(x, axis=0)
    sh = x - m[None]
    s = jnp.sum(jnp.exp(sh), axis=0)
    st = jnp.zeros_like(m)
    wt = jnp.zeros_like(m)
    for c in range(_C):
        sel = t == c
        st = jnp.where(sel, sh[c], st)
        wt = jnp.where(sel, jnp.float32(_W[c]), wt)
    return st, jnp.log(s), wt


_NEG_LOG_THRESH = 0.35667494393873245


def _fused_body(pred_ref, tgt_ref, cnt_ref, a_ref, b_ref):
    x = pred_ref[0]
    t = tgt_ref[0]
    st, logs, wt = _pixel_stats(x, t)
    nll = logs - st
    kept = nll >= jnp.float32(_NEG_LOG_THRESH)
    cnt_p = jnp.sum(kept.astype(jnp.float32))
    a_p = jnp.sum(jnp.where(kept, wt * nll, 0.0))
    b_p = jnp.sum(jnp.where(kept, wt, 0.0))

    first = jnp.logical_and(pl.program_id(0) == 0, pl.program_id(1) == 0)

    @pl.when(first)
    def _init():
        cnt_ref[0, 0] = jnp.float32(0)
        a_ref[0, 0] = jnp.float32(0)
        b_ref[0, 0] = jnp.float32(0)

    cnt_ref[0, 0] += cnt_p
    a_ref[0, 0] += a_p
    b_ref[0, 0] += b_p


def _fallback_arrays_body(pred_ref, tgt_ref, p_ref, nll_ref, wt_ref):
    x = pred_ref[0]
    t = tgt_ref[0]
    st, logs, wt = _pixel_stats(x, t)
    p_ref[0] = jnp.exp(st - logs)
    nll_ref[0] = logs - st
    wt_ref[0] = wt


def _fallback_reduce_body(p_ref, nll_ref, wt_ref, thr_ref, num_ref, den_ref):
    p = p_ref[...]
    u = lax.bitcast_convert_type(p, jnp.int32)

    def step(_, carry):
        lo, hi = carry
        mid = (lo + hi) // 2
        c = jnp.sum((u <= mid).astype(jnp.float32))
        take = c >= jnp.float32(_MIN_KEPT)
        return jnp.where(take, lo, mid + 1), jnp.where(take, mid, hi)

    lo0 = jnp.int32(0)
    hi0 = jnp.int32(0x7F800000)
    _, hi = lax.fori_loop(0, 31, step, (lo0, hi0))
    tval = lax.bitcast_convert_type(hi, jnp.float32)
    thr = jnp.maximum(tval, jnp.float32(_THRESH))
    kept = p <= thr
    w = jnp.where(kept, wt_ref[...], 0.0)
    num_ref[0, 0] = jnp.sum(w * nll_ref[...])
    den_ref[0, 0] = jnp.sum(w)
    thr_ref[0, 0] = thr


def _hot_path(pred, tgt):
    n, c, h, w = pred.shape
    grid = (n, h // _BH)
    out = pl.pallas_call(
        _fused_body,
        grid=grid,
        in_specs=[
            pl.BlockSpec((1, c, _BH, w), lambda i, j: (i, 0, j, 0)),
            pl.BlockSpec((1, _BH, w), lambda i, j: (i, j, 0)),
        ],
        out_specs=[
            pl.BlockSpec((1, 1), lambda i, j: (0, 0),
                         memory_space=pltpu.SMEM),
            pl.BlockSpec((1, 1), lambda i, j: (0, 0),
                         memory_space=pltpu.SMEM),
            pl.BlockSpec((1, 1), lambda i, j: (0, 0),
                         memory_space=pltpu.SMEM),
        ],
        out_shape=[jax.ShapeDtypeStruct((1, 1), jnp.float32)] * 3,
    )(pred, tgt)
    return out[0][0, 0], out[1][0, 0], out[2][0, 0]


def _fallback(pred, tgt):
    n, c, h, w = pred.shape
    grid = (n, h // _BH)
    px_shape = jax.ShapeDtypeStruct((n, h, w), jnp.float32)
    p, nll, wt = pl.pallas_call(
        _fallback_arrays_body,
        grid=grid,
        in_specs=[
            pl.BlockSpec((1, c, _BH, w), lambda i, j: (i, 0, j, 0)),
            pl.BlockSpec((1, _BH, w), lambda i, j: (i, j, 0)),
        ],
        out_specs=[pl.BlockSpec((1, _BH, w), lambda i, j: (i, j, 0))] * 3,
        out_shape=[px_shape] * 3,
    )(pred, tgt)
    thr, num, den = pl.pallas_call(
        _fallback_reduce_body,
        out_specs=[
            pl.BlockSpec(memory_space=pltpu.SMEM),
            pl.BlockSpec(memory_space=pltpu.SMEM),
            pl.BlockSpec(memory_space=pltpu.SMEM),
        ],
        out_shape=[jax.ShapeDtypeStruct((1, 1), jnp.float32)] * 3,
    )(p, nll, wt)
    return num[0, 0], den[0, 0]


@jax.jit
def kernel(inputs, targets):
    pred = inputs.astype(jnp.float32)
    tgt = targets.astype(jnp.int32)
    cnt, a, b = _hot_path(pred, tgt)
    num, den = lax.cond(
        cnt >= jnp.float32(_MIN_KEPT),
        lambda: (a, b),
        lambda: _fallback(pred, tgt),
    )
    return num / den
